# Initial kernel scaffold; baseline (speedup 1.0000x reference)
import jax
import jax.numpy as jnp
from jax.experimental import pallas as pl


def kernel(logits, temperature, top_k, top_p):
    raise NotImplementedError("write your pallas kernel here")


# speedup vs baseline: 14.0187x; 14.0187x over previous
import jax
import jax.numpy as jnp
from jax import lax
from jax.experimental import pallas as pl

_EPS = 1e-5
_NLP = 20
_ROWS = 8


def _f2k(f):
    b = lax.bitcast_convert_type(f, jnp.int32)
    return b ^ (lax.shift_right_arithmetic(b, 31) & jnp.int32(0x7FFFFFFF))


def _bsearch(keys, pred, iters=34):
    lo = jnp.min(keys, axis=1, keepdims=True) - 1
    hi = jnp.max(keys, axis=1, keepdims=True)

    def body(_, c):
        lo, hi = c
        mid = (lo >> 1) + (hi >> 1) + (lo & hi & 1)
        p = pred(mid)
        return jnp.where(p, lo, mid), jnp.where(p, mid, hi)

    lo, hi = lax.fori_loop(0, iters, body, (lo, hi))
    return hi


def _make_body(V):
    def body(y_ref, g_ref, t_ref, tk_ref, tp_ref, samp_ref, vals_ref, idx_ref):
        y = y_ref[...]
        R, Vp = y.shape
        iota = lax.broadcasted_iota(jnp.int32, (R, Vp), 1)
        temp_raw = t_ref[...]
        tk = tk_ref[...]
        tp = tp_ref[...]

        my = jnp.max(y, axis=1, keepdims=True)
        sh_lse = jnp.log(jnp.sum(jnp.exp(y - my), axis=1, keepdims=True))

        temp = jnp.where(temp_raw < _EPS, jnp.float32(1.0), temp_raw)
        x = y / temp
        xk = _f2k(x)

        keff = jnp.where(tk <= 0, jnp.int32(V), jnp.minimum(tk, jnp.int32(V)))

        def pred_k(mid):
            cnt = jnp.sum((xk > mid).astype(jnp.int32), axis=1, keepdims=True)
            return cnt <= keff - 1

        kth_key = _bsearch(xk, pred_k)

        mx = jnp.max(x, axis=1, keepdims=True)
        e = jnp.where(xk >= kth_key, jnp.exp(x - mx), jnp.float32(0.0))
        zden = jnp.sum(e, axis=1, keepdims=True)
        mass_lim = tp * zden

        def pred_p(mid):
            s = jnp.sum(jnp.where(xk > mid, e, jnp.float32(0.0)),
                        axis=1, keepdims=True)
            return s <= mass_lim

        cut_key = _bsearch(xk, pred_p)

        z = jnp.where(xk >= cut_key, x + g_ref[...], -jnp.inf)
        mz = jnp.max(z, axis=1, keepdims=True)
        samp = jnp.min(jnp.where(z == mz, iota, jnp.int32(Vp)),
                       axis=1, keepdims=True)

        curval = jnp.full((R, 1), jnp.inf, jnp.float32)
        curidx = jnp.full((R, 1), -1, jnp.int32)
        tvals, tidxs = [], []
        for _ in range(_NLP):
            ok = (y < curval) | ((y == curval) & (iota > curidx))
            v = jnp.where(ok, y, -jnp.inf)
            m = jnp.max(v, axis=1, keepdims=True)
            ix = jnp.min(jnp.where(v == m, iota, jnp.int32(Vp)),
                         axis=1, keepdims=True)
            tvals.append(m)
            tidxs.append(ix)
            curval, curidx = m, ix

        greedy = tidxs[0]
        sampled = jnp.where(temp_raw < _EPS, greedy, samp)
        ysamp = jnp.sum(jnp.where(iota == sampled, y, jnp.float32(0.0)),
                        axis=1, keepdims=True)

        samp_ref[...] = sampled
        vals_ref[...] = jnp.concatenate(
            [(ysamp - my) - sh_lse] + [(m - my) - sh_lse for m in tvals],
            axis=1)
        idx_ref[...] = jnp.concatenate([sampled] + tidxs, axis=1)

    return body


def kernel(logits, temperature, top_k, top_p):
    B, V = logits.shape
    logits = logits.astype(jnp.float32)
    Vp = ((V + 127) // 128) * 128
    gumbel = jax.random.gumbel(jax.random.key(1234), (B, V), jnp.float32)
    ypad = jnp.pad(logits, ((0, 0), (0, Vp - V)), constant_values=-jnp.inf)
    gpad = jnp.pad(gumbel, ((0, 0), (0, Vp - V)), constant_values=0.0)
    t2 = temperature.astype(jnp.float32).reshape(B, 1)
    tk2 = top_k.astype(jnp.int32).reshape(B, 1)
    tp2 = top_p.astype(jnp.float32).reshape(B, 1)

    R = _ROWS
    grid = (B // R,)
    sampled, vals, idx = pl.pallas_call(
        _make_body(V),
        grid=grid,
        in_specs=[
            pl.BlockSpec((R, Vp), lambda i: (i, 0)),
            pl.BlockSpec((R, Vp), lambda i: (i, 0)),
            pl.BlockSpec((R, 1), lambda i: (i, 0)),
            pl.BlockSpec((R, 1), lambda i: (i, 0)),
            pl.BlockSpec((R, 1), lambda i: (i, 0)),
        ],
        out_specs=[
            pl.BlockSpec((R, 1), lambda i: (i, 0)),
            pl.BlockSpec((R, _NLP + 1), lambda i: (i, 0)),
            pl.BlockSpec((R, _NLP + 1), lambda i: (i, 0)),
        ],
        out_shape=[
            jax.ShapeDtypeStruct((B, 1), jnp.int32),
            jax.ShapeDtypeStruct((B, _NLP + 1), jnp.float32),
            jax.ShapeDtypeStruct((B, _NLP + 1), jnp.int32),
        ],
    )(ypad, gpad, t2, tk2, tp2)
    return sampled, vals, idx


# speedup vs baseline: 14.7380x; 1.0513x over previous
import jax
import jax.numpy as jnp
from jax import lax
from jax.experimental import pallas as pl

_EPS = 1e-5
_NLP = 20
_ROWS = 8


def _f2k(f):
    b = lax.bitcast_convert_type(f, jnp.int32)
    return b ^ (lax.shift_right_arithmetic(b, 31) & jnp.int32(0x7FFFFFFF))


def _bsearch(keys, pred, iters=34):
    lo = jnp.min(keys, axis=1, keepdims=True) - 1
    hi = jnp.max(keys, axis=1, keepdims=True)

    def body(_, c):
        lo, hi = c
        mid = (lo >> 1) + (hi >> 1) + (lo & hi & 1)
        p = pred(mid)
        return jnp.where(p, lo, mid), jnp.where(p, mid, hi)

    lo, hi = lax.fori_loop(0, iters, body, (lo, hi))
    return hi


def _make_body(V):
    def body(y_ref, g_ref, t_ref, tk_ref, tp_ref, samp_ref, vals_ref, idx_ref):
        y = y_ref[...]
        R, Vp = y.shape
        iota = lax.broadcasted_iota(jnp.int32, (R, Vp), 1)
        temp_raw = t_ref[...]
        tk = tk_ref[...]
        tp = tp_ref[...]

        my = jnp.max(y, axis=1, keepdims=True)
        sh_lse = jnp.log(jnp.sum(jnp.exp(y - my), axis=1, keepdims=True))

        temp = jnp.where(temp_raw < _EPS, jnp.float32(1.0), temp_raw)
        x = y / temp
        xk = _f2k(x)

        keff = jnp.where(tk <= 0, jnp.int32(V), jnp.minimum(tk, jnp.int32(V)))

        def pred_k(mid):
            cnt = jnp.sum((xk > mid).astype(jnp.int32), axis=1, keepdims=True)
            return cnt <= keff - 1

        kth_key = _bsearch(xk, pred_k)

        mx = jnp.max(x, axis=1, keepdims=True)
        e = jnp.where(xk >= kth_key, jnp.exp(x - mx), jnp.float32(0.0))
        zden = jnp.sum(e, axis=1, keepdims=True)
        mass_lim = tp * zden

        def pred_p(mid):
            s = jnp.sum(jnp.where(xk > mid, e, jnp.float32(0.0)),
                        axis=1, keepdims=True)
            return s <= mass_lim

        cut_key = _bsearch(xk, pred_p)

        z = jnp.where(xk >= cut_key, x + g_ref[...], -jnp.inf)
        mz = jnp.max(z, axis=1, keepdims=True)
        samp = jnp.min(jnp.where(z == mz, iota, jnp.int32(Vp)),
                       axis=1, keepdims=True)

        y_work = y
        tvals, tidxs = [], []
        for _ in range(_NLP):
            m = jnp.max(y_work, axis=1, keepdims=True)
            ix = jnp.min(jnp.where(y_work == m, iota, jnp.int32(Vp)),
                         axis=1, keepdims=True)
            tvals.append(m)
            tidxs.append(ix)
            y_work = jnp.where(iota == ix, -jnp.inf, y_work)

        greedy = tidxs[0]
        sampled = jnp.where(temp_raw < _EPS, greedy, samp)
        ysamp = jnp.sum(jnp.where(iota == sampled, y, jnp.float32(0.0)),
                        axis=1, keepdims=True)

        samp_ref[...] = sampled
        vals_ref[...] = jnp.concatenate(
            [(ysamp - my) - sh_lse] + [(m - my) - sh_lse for m in tvals],
            axis=1)
        idx_ref[...] = jnp.concatenate([sampled] + tidxs, axis=1)

    return body


def kernel(logits, temperature, top_k, top_p):
    B, V = logits.shape
    logits = logits.astype(jnp.float32)
    Vp = ((V + 127) // 128) * 128
    gumbel = jax.random.gumbel(jax.random.key(1234), (B, V), jnp.float32)
    ypad = jnp.pad(logits, ((0, 0), (0, Vp - V)), constant_values=-jnp.inf)
    gpad = jnp.pad(gumbel, ((0, 0), (0, Vp - V)), constant_values=0.0)
    t2 = temperature.astype(jnp.float32).reshape(B, 1)
    tk2 = top_k.astype(jnp.int32).reshape(B, 1)
    tp2 = top_p.astype(jnp.float32).reshape(B, 1)

    R = _ROWS
    grid = (B // R,)
    sampled, vals, idx = pl.pallas_call(
        _make_body(V),
        grid=grid,
        in_specs=[
            pl.BlockSpec((R, Vp), lambda i: (i, 0)),
            pl.BlockSpec((R, Vp), lambda i: (i, 0)),
            pl.BlockSpec((R, 1), lambda i: (i, 0)),
            pl.BlockSpec((R, 1), lambda i: (i, 0)),
            pl.BlockSpec((R, 1), lambda i: (i, 0)),
        ],
        out_specs=[
            pl.BlockSpec((R, 1), lambda i: (i, 0)),
            pl.BlockSpec((R, _NLP + 1), lambda i: (i, 0)),
            pl.BlockSpec((R, _NLP + 1), lambda i: (i, 0)),
        ],
        out_shape=[
            jax.ShapeDtypeStruct((B, 1), jnp.int32),
            jax.ShapeDtypeStruct((B, _NLP + 1), jnp.float32),
            jax.ShapeDtypeStruct((B, _NLP + 1), jnp.int32),
        ],
    )(ypad, gpad, t2, tk2, tp2)
    return sampled, vals, idx


# speedup vs baseline: 15.0310x; 1.0199x over previous
import jax
import jax.numpy as jnp
from jax import lax
from jax.experimental import pallas as pl

_EPS = 1e-5
_NLP = 20
_ROWS = 8


def _f2k(f):
    b = lax.bitcast_convert_type(f, jnp.int32)
    return b ^ (lax.shift_right_arithmetic(b, 31) & jnp.int32(0x7FFFFFFF))


def _bsearch(keys, pred3, iters=19):
    lo = jnp.min(keys, axis=1, keepdims=True) - 1
    hi = jnp.max(keys, axis=1, keepdims=True)

    def body(_, c):
        lo, hi = c
        ln = hi - lo
        q1 = lax.shift_right_logical(ln, 2)
        q2 = lax.shift_right_logical(ln, 1)
        m1 = lo + q1
        m2 = lo + q2
        m3 = lo + q1 + q2
        p1, p2, p3 = pred3(m1, m2, m3)
        nlo = jnp.where(p1, lo, jnp.where(p2, m1, jnp.where(p3, m2, m3)))
        nhi = jnp.where(p1, m1, jnp.where(p2, m2, jnp.where(p3, m3, hi)))
        return nlo, nhi

    lo, hi = lax.fori_loop(0, iters, body, (lo, hi))
    return hi


def _make_body(V):
    def body(y_ref, g_ref, t_ref, tk_ref, tp_ref, samp_ref, vals_ref, idx_ref):
        y = y_ref[...]
        R, Vp = y.shape
        iota = lax.broadcasted_iota(jnp.int32, (R, Vp), 1)
        temp_raw = t_ref[...]
        tk = tk_ref[...]
        tp = tp_ref[...]

        my = jnp.max(y, axis=1, keepdims=True)
        sh_lse = jnp.log(jnp.sum(jnp.exp(y - my), axis=1, keepdims=True))

        temp = jnp.where(temp_raw < _EPS, jnp.float32(1.0), temp_raw)
        x = y / temp
        xk = _f2k(x)

        keff = jnp.where(tk <= 0, jnp.int32(V), jnp.minimum(tk, jnp.int32(V)))

        def pred_k(m1, m2, m3):
            lim = keff - 1
            c1 = jnp.sum((xk > m1).astype(jnp.int32), axis=1, keepdims=True)
            c2 = jnp.sum((xk > m2).astype(jnp.int32), axis=1, keepdims=True)
            c3 = jnp.sum((xk > m3).astype(jnp.int32), axis=1, keepdims=True)
            return c1 <= lim, c2 <= lim, c3 <= lim

        kth_key = _bsearch(xk, pred_k)

        mx = jnp.max(x, axis=1, keepdims=True)
        e = jnp.where(xk >= kth_key, jnp.exp(x - mx), jnp.float32(0.0))
        zden = jnp.sum(e, axis=1, keepdims=True)
        mass_lim = tp * zden

        def pred_p(m1, m2, m3):
            z0 = jnp.float32(0.0)
            s1 = jnp.sum(jnp.where(xk > m1, e, z0), axis=1, keepdims=True)
            s2 = jnp.sum(jnp.where(xk > m2, e, z0), axis=1, keepdims=True)
            s3 = jnp.sum(jnp.where(xk > m3, e, z0), axis=1, keepdims=True)
            return s1 <= mass_lim, s2 <= mass_lim, s3 <= mass_lim

        cut_key = _bsearch(xk, pred_p)

        z = jnp.where(xk >= cut_key, x + g_ref[...], -jnp.inf)
        mz = jnp.max(z, axis=1, keepdims=True)
        samp = jnp.min(jnp.where(z == mz, iota, jnp.int32(Vp)),
                       axis=1, keepdims=True)

        y_work = y
        tvals, tidxs = [], []
        for _ in range(_NLP):
            m = jnp.max(y_work, axis=1, keepdims=True)
            ix = jnp.min(jnp.where(y_work == m, iota, jnp.int32(Vp)),
                         axis=1, keepdims=True)
            tvals.append(m)
            tidxs.append(ix)
            y_work = jnp.where(iota == ix, -jnp.inf, y_work)

        greedy = tidxs[0]
        sampled = jnp.where(temp_raw < _EPS, greedy, samp)
        ysamp = jnp.sum(jnp.where(iota == sampled, y, jnp.float32(0.0)),
                        axis=1, keepdims=True)

        samp_ref[...] = sampled
        vals_ref[...] = jnp.concatenate(
            [(ysamp - my) - sh_lse] + [(m - my) - sh_lse for m in tvals],
            axis=1)
        idx_ref[...] = jnp.concatenate([sampled] + tidxs, axis=1)

    return body


def kernel(logits, temperature, top_k, top_p):
    B, V = logits.shape
    logits = logits.astype(jnp.float32)
    Vp = ((V + 127) // 128) * 128
    gumbel = jax.random.gumbel(jax.random.key(1234), (B, V), jnp.float32)
    ypad = jnp.pad(logits, ((0, 0), (0, Vp - V)), constant_values=-jnp.inf)
    gpad = jnp.pad(gumbel, ((0, 0), (0, Vp - V)), constant_values=0.0)
    t2 = temperature.astype(jnp.float32).reshape(B, 1)
    tk2 = top_k.astype(jnp.int32).reshape(B, 1)
    tp2 = top_p.astype(jnp.float32).reshape(B, 1)

    R = _ROWS
    grid = (B // R,)
    sampled, vals, idx = pl.pallas_call(
        _make_body(V),
        grid=grid,
        in_specs=[
            pl.BlockSpec((R, Vp), lambda i: (i, 0)),
            pl.BlockSpec((R, Vp), lambda i: (i, 0)),
            pl.BlockSpec((R, 1), lambda i: (i, 0)),
            pl.BlockSpec((R, 1), lambda i: (i, 0)),
            pl.BlockSpec((R, 1), lambda i: (i, 0)),
        ],
        out_specs=[
            pl.BlockSpec((R, 1), lambda i: (i, 0)),
            pl.BlockSpec((R, _NLP + 1), lambda i: (i, 0)),
            pl.BlockSpec((R, _NLP + 1), lambda i: (i, 0)),
        ],
        out_shape=[
            jax.ShapeDtypeStruct((B, 1), jnp.int32),
            jax.ShapeDtypeStruct((B, _NLP + 1), jnp.float32),
            jax.ShapeDtypeStruct((B, _NLP + 1), jnp.int32),
        ],
    )(ypad, gpad, t2, tk2, tp2)
    return sampled, vals, idx
